# baseline (speedup 1.0000x reference)
import jax
import jax.numpy as jnp
from jax.experimental import pallas as pl
from jax.experimental.pallas import tpu as pltpu

N = 4096
T = 16
BM = 256
SPLIT = 16
BK = N // SPLIT


def _fused_kernel(*refs):
    xi, xj1, xk, wi, wj1, wj2, wk = refs[:7]
    g_refs = refs[7:7 + 4 * SPLIT]
    out = refs[7 + 4 * SPLIT]
    ys = refs[8 + 4 * SPLIT:]
    bf = jnp.bfloat16

    @pl.when(pl.program_id(0) == 0)
    def _compute_projections():
        for y, x, w in ((ys[0], xi, wi), (ys[1], xj1, wj1),
                        (ys[2], xj1, wj2), (ys[3], xk, wk)):
            y[...] = jnp.dot(
                x[...], w[...], preferred_element_type=jnp.float32
            ).astype(bf)

    acc = jnp.zeros((BM, T), dtype=jnp.float32)
    for m in range(4):
        for s in range(SPLIT):
            g = g_refs[m * SPLIT + s]
            y = ys[m][s * BK:(s + 1) * BK, :]
            acc += jnp.dot(g[...].astype(bf), y,
                           preferred_element_type=jnp.float32)
    out[...] = jnp.maximum(acc, 0.0)


@jax.jit
def kernel(xi, xj1, xj2, xk, Gi2j, Adj2j, coAdj2j, Gk2j, W_i, W_j1, W_j2, W_k):
    del xj2

    grid = (N // BM,)
    feat_spec = pl.BlockSpec((N, T), lambda i: (0, 0))
    w_spec = pl.BlockSpec((T, T), lambda i: (0, 0))

    def col_spec(s):
        return pl.BlockSpec((BM, BK), lambda i, s=s: (i, s))

    g_specs = []
    g_args = []
    for G in (Gi2j, Adj2j, coAdj2j, Gk2j):
        for s in range(SPLIT):
            g_specs.append(col_spec(s))
            g_args.append(G)

    y_scratch = pltpu.VMEM((N, T), jnp.bfloat16)
    out = pl.pallas_call(
        _fused_kernel,
        grid=grid,
        in_specs=[feat_spec, feat_spec, feat_spec,
                  w_spec, w_spec, w_spec, w_spec] + g_specs,
        out_specs=pl.BlockSpec((BM, T), lambda i: (i, 0)),
        out_shape=jax.ShapeDtypeStruct((N, T), jnp.float32),
        scratch_shapes=[y_scratch, y_scratch, y_scratch, y_scratch],
        compiler_params=pltpu.CompilerParams(
            dimension_semantics=("arbitrary",),
        ),
    )(xi, xj1, xk, W_i, W_j1, W_j2, W_k, *g_args)
    return out


# speedup vs baseline: 1.0253x; 1.0253x over previous
import jax
import jax.numpy as jnp
from jax.experimental import pallas as pl
from jax.experimental.pallas import tpu as pltpu

N = 4096
T = 16
BM = 128
SPLIT = 8
BK = N // SPLIT


def _fused_kernel(*refs):
    xi, xj1, xk, wi, wj1, wj2, wk = refs[:7]
    g_refs = refs[7:7 + 4 * SPLIT]
    out = refs[7 + 4 * SPLIT]
    ys = refs[8 + 4 * SPLIT:]
    bf = jnp.bfloat16

    @pl.when(pl.program_id(0) == 0)
    def _compute_projections():
        for y, x, w in ((ys[0], xi, wi), (ys[1], xj1, wj1),
                        (ys[2], xj1, wj2), (ys[3], xk, wk)):
            y[...] = jnp.dot(
                x[...], w[...], preferred_element_type=jnp.float32
            ).astype(bf)

    acc = jnp.zeros((BM, T), dtype=jnp.float32)
    for m in range(4):
        for s in range(SPLIT):
            g = g_refs[m * SPLIT + s]
            y = ys[m][s * BK:(s + 1) * BK, :]
            acc += jnp.dot(g[...].astype(bf), y,
                           preferred_element_type=jnp.float32)
    out[...] = jnp.maximum(acc, 0.0)


@jax.jit
def kernel(xi, xj1, xj2, xk, Gi2j, Adj2j, coAdj2j, Gk2j, W_i, W_j1, W_j2, W_k):
    del xj2

    grid = (N // BM,)
    feat_spec = pl.BlockSpec((N, T), lambda i: (0, 0))
    w_spec = pl.BlockSpec((T, T), lambda i: (0, 0))

    def col_spec(s):
        return pl.BlockSpec((BM, BK), lambda i, s=s: (i, s))

    g_specs = []
    g_args = []
    for G in (Gi2j, Adj2j, coAdj2j, Gk2j):
        for s in range(SPLIT):
            g_specs.append(col_spec(s))
            g_args.append(G)

    y_scratch = pltpu.VMEM((N, T), jnp.bfloat16)
    out = pl.pallas_call(
        _fused_kernel,
        grid=grid,
        in_specs=[feat_spec, feat_spec, feat_spec,
                  w_spec, w_spec, w_spec, w_spec] + g_specs,
        out_specs=pl.BlockSpec((BM, T), lambda i: (i, 0)),
        out_shape=jax.ShapeDtypeStruct((N, T), jnp.float32),
        scratch_shapes=[y_scratch, y_scratch, y_scratch, y_scratch],
        compiler_params=pltpu.CompilerParams(
            dimension_semantics=("arbitrary",),
        ),
    )(xi, xj1, xk, W_i, W_j1, W_j2, W_k, *g_args)
    return out
